# Initial kernel scaffold; baseline (speedup 1.0000x reference)
import jax
import jax.numpy as jnp
from jax.experimental import pallas as pl


def kernel(x, edge_index, edge_attr, batch, pharma_index, ew, eb, g1_Wl, g1_bl, g1_Wr, g1_br, g1_att, g1_We, g1_bias, g2_Wl, g2_bl, g2_Wr, g2_br, g2_att, g2_We, g2_bias, g3_Wl, g3_bl, g3_Wr, g3_br, g3_att, g3_bias, g4_Wl, g4_bl, g4_Wr, g4_br, g4_att, g4_bias, l1_W, l1_b, l2_W, l2_b):
    raise NotImplementedError("write your pallas kernel here")


# speedup vs baseline: 3.0184x; 3.0184x over previous
import jax
import jax.numpy as jnp
from jax.experimental import pallas as pl

_EPS = 1e-16


def _mm(x, wt, b, bm=512):
    m, k = x.shape
    mo = wt.shape[1]
    bm = min(bm, max(8, (m + 7) // 8 * 8))

    def kern(xr, wr, br, orr):
        orr[...] = jnp.dot(xr[...], wr[...],
                           preferred_element_type=jnp.float32) + br[...]

    return pl.pallas_call(
        kern,
        grid=(pl.cdiv(m, bm),),
        in_specs=[
            pl.BlockSpec((bm, k), lambda i: (i, 0)),
            pl.BlockSpec((k, mo), lambda i: (0, 0)),
            pl.BlockSpec((1, mo), lambda i: (0, 0)),
        ],
        out_specs=pl.BlockSpec((bm, mo), lambda i: (i, 0)),
        out_shape=jax.ShapeDtypeStruct((m, mo), jnp.float32),
    )(x, wt, b.reshape(1, -1))


def _edge_stage(gl, gr, ge, att, hmap, hmapt, bm=512):
    ne, c = gl.shape
    has_ge = ge is not None

    def kern(*refs):
        if has_ge:
            glr, grr, ger, ar, hr, htr, wr, pr = refs
            msg = glr[...] + grr[...] + ger[...]
        else:
            glr, grr, ar, hr, htr, wr, pr = refs
            msg = glr[...] + grr[...]
        t = jnp.where(msg > 0, msg, 0.2 * msg) * ar[...]
        logits = jnp.dot(t, hr[...], preferred_element_type=jnp.float32)
        p = jnp.exp(logits)
        pe = jnp.dot(p, htr[...], preferred_element_type=jnp.float32)
        wr[...] = glr[...] * pe
        pr[...] = p

    row = pl.BlockSpec((bm, c), lambda i: (i, 0))
    in_specs = [row, row]
    args = [gl, gr]
    if has_ge:
        in_specs.append(row)
        args.append(ge)
    in_specs += [
        pl.BlockSpec((1, c), lambda i: (0, 0)),
        pl.BlockSpec((c, 128), lambda i: (0, 0)),
        pl.BlockSpec((128, c), lambda i: (0, 0)),
    ]
    args += [att, hmap, hmapt]
    return pl.pallas_call(
        kern,
        grid=(pl.cdiv(ne, bm),),
        in_specs=in_specs,
        out_specs=[
            pl.BlockSpec((bm, c), lambda i: (i, 0)),
            pl.BlockSpec((bm, 128), lambda i: (i, 0)),
        ],
        out_shape=[
            jax.ShapeDtypeStruct((ne, c), jnp.float32),
            jax.ShapeDtypeStruct((ne, 128), jnp.float32),
        ],
    )(*args)


def _normalize(num, s_pad, hmapt, bias, bm=512):
    n, c = num.shape

    def kern(nr, sr, htr, br, orr):
        sexp = jnp.dot(sr[...], htr[...], preferred_element_type=jnp.float32)
        o = nr[...] / (sexp + _EPS) + br[...]
        orr[...] = jnp.where(o > 0, o, jnp.exp(jnp.minimum(o, 0.0)) - 1.0)

    return pl.pallas_call(
        kern,
        grid=(pl.cdiv(n, bm),),
        in_specs=[
            pl.BlockSpec((bm, c), lambda i: (i, 0)),
            pl.BlockSpec((bm, 128), lambda i: (i, 0)),
            pl.BlockSpec((128, c), lambda i: (0, 0)),
            pl.BlockSpec((1, c), lambda i: (0, 0)),
        ],
        out_specs=pl.BlockSpec((bm, c), lambda i: (i, 0)),
        out_shape=jax.ShapeDtypeStruct((n, c), jnp.float32),
    )(num, s_pad, hmapt, bias.reshape(1, -1))


def _scatter_mean(data, idx, num_segments):
    s = jax.ops.segment_sum(data, idx, num_segments=num_segments)
    c = jax.ops.segment_sum(jnp.ones((idx.shape[0],), dtype=data.dtype), idx,
                            num_segments=num_segments)
    c = jnp.clip(c, 1.0, None)
    if data.ndim > 1:
        c = c.reshape((-1,) + (1,) * (data.ndim - 1))
    return s / c


def _gat_layer(x, src, dst, wl, bl, wr, br, att, bias, heads, out_ch,
               edge_feat=None, we=None):
    n = x.shape[0]
    c = heads * out_ch
    loop = jnp.arange(n, dtype=src.dtype)
    if edge_feat is not None:
        loop_attr = _scatter_mean(edge_feat, dst, n)
        edge_feat = jnp.concatenate([edge_feat, loop_attr], axis=0)
    src2 = jnp.concatenate([src, loop])
    dst2 = jnp.concatenate([dst, loop])

    xl = _mm(x, wl.T, bl)
    xr = _mm(x, wr.T, br)
    ge = _mm(edge_feat, we.T, jnp.zeros((c,), jnp.float32)) \
        if edge_feat is not None else None

    gl = xl[src2]
    gr = xr[dst2]

    hmap = (jnp.arange(c)[:, None] // out_ch
            == jnp.arange(128)[None, :]).astype(jnp.float32)
    hmapt = hmap.T
    att1 = att.reshape(1, c)

    w, p_pad = _edge_stage(gl, gr, ge, att1, hmap, hmapt)
    p = p_pad[:, :heads]
    s = jax.ops.segment_sum(p, dst2, num_segments=n)
    num = jax.ops.segment_sum(w, dst2, num_segments=n)
    s_pad = jnp.pad(s, ((0, 0), (0, 128 - heads)))
    return _normalize(num, s_pad, hmapt, bias)


def _rg_edges(edge_index, group_idx, num_groups):
    ge = group_idx[edge_index]
    valid = ge[0] != ge[1]
    sent = num_groups * num_groups
    key = jnp.where(valid, ge[0] * num_groups + ge[1], sent)
    key = jnp.sort(key)
    first = jnp.concatenate([jnp.ones((1,), dtype=jnp.bool_),
                             key[1:] != key[:-1]])
    keep = first & (key < sent)
    srcg = jnp.where(keep, key // num_groups, num_groups)
    dstg = jnp.where(keep, key % num_groups, num_groups)
    return jnp.stack([srcg, dstg], axis=0), jnp.any(keep)


def _mlp_head(pooled, w1t, b1, w2t_pad, b2_pad):
    def kern(pr, w1r, b1r, w2r, b2r, orr):
        o = jnp.dot(pr[...], w1r[...], preferred_element_type=jnp.float32)
        o = jnp.maximum(o + b1r[...], 0.0)
        orr[...] = jnp.dot(o, w2r[...],
                           preferred_element_type=jnp.float32) + b2r[...]

    m, k = pooled.shape
    ko = w1t.shape[1]
    return pl.pallas_call(
        kern,
        grid=(1,),
        in_specs=[
            pl.BlockSpec((m, k), lambda i: (0, 0)),
            pl.BlockSpec((k, ko), lambda i: (0, 0)),
            pl.BlockSpec((1, ko), lambda i: (0, 0)),
            pl.BlockSpec((ko, 128), lambda i: (0, 0)),
            pl.BlockSpec((1, 128), lambda i: (0, 0)),
        ],
        out_specs=pl.BlockSpec((m, 128), lambda i: (0, 0)),
        out_shape=jax.ShapeDtypeStruct((m, 128), jnp.float32),
    )(pooled, w1t, b1.reshape(1, -1), w2t_pad, b2_pad.reshape(1, -1))


def kernel(x, edge_index, edge_attr, batch, pharma_index, ew, eb, g1_Wl, g1_bl, g1_Wr, g1_br, g1_att, g1_We, g1_bias, g2_Wl, g2_bl, g2_Wr, g2_br, g2_att, g2_We, g2_bias, g3_Wl, g3_bl, g3_Wr, g3_br, g3_att, g3_bias, g4_Wl, g4_bl, g4_Wr, g4_br, g4_att, g4_bias, l1_W, l1_b, l2_W, l2_b):
    n = x.shape[0]
    g = 2000
    b_sz = 64
    h = 64

    src = edge_index[0]
    dst = edge_index[1]
    ea = _mm(edge_attr, ew.T, eb)

    h1 = _gat_layer(x, src, dst, g1_Wl, g1_bl, g1_Wr, g1_br, g1_att, g1_bias,
                    4, h, ea, g1_We)
    h2 = _gat_layer(h1, src, dst, g2_Wl, g2_bl, g2_Wr, g2_br, g2_att, g2_bias,
                    1, h, ea, g2_We)

    grouped = _scatter_mean(h2, pharma_index, g)
    red_batch = _scatter_mean(batch.astype(jnp.float32), pharma_index,
                              g).astype(jnp.int32)
    nei, has_edges = _rg_edges(edge_index, pharma_index, g)

    def with_edges(gz):
        z1 = _gat_layer(gz, nei[0], nei[1], g3_Wl, g3_bl, g3_Wr, g3_br,
                        g3_att, g3_bias, 1, h)
        return _gat_layer(z1, nei[0], nei[1], g4_Wl, g4_bl, g4_Wr, g4_br,
                          g4_att, g4_bias, 1, h)

    z = jax.lax.cond(has_edges, with_edges, lambda gz: gz, grouped)
    pooled = _scatter_mean(z, red_batch, b_sz)

    w2t_pad = jnp.pad(l2_W.T, ((0, 0), (0, 127)))
    b2_pad = jnp.pad(l2_b, (0, 127))
    out = _mlp_head(pooled, l1_W.T, l1_b, w2t_pad, b2_pad)
    return out[:, :1]


# speedup vs baseline: 3.7777x; 1.2516x over previous
import functools

import jax
import jax.numpy as jnp
from jax import lax
from jax.experimental import pallas as pl
from jax.experimental.pallas import tpu as pltpu
from jax.experimental.pallas import tpu_sc as plsc

_EPS = 1e-16
_CH = 128


def _sc_pair_gather(xl, xr, sidx, didx):
    d = xl.shape[1]
    bpad = sidx.shape[0]
    info = plsc.get_sparse_core_info()
    nc = info.num_cores
    nw = nc * info.num_subcores
    bw = bpad // nw
    chunks = bw // _CH
    mesh = plsc.VectorSubcoreMesh(core_axis_name="c", subcore_axis_name="s")

    @functools.partial(
        pl.kernel,
        mesh=mesh,
        out_type=[
            jax.ShapeDtypeStruct((bpad, d), jnp.float32),
            jax.ShapeDtypeStruct((bpad, d), jnp.float32),
        ],
        scratch_types=[
            pltpu.VMEM((_CH,), jnp.int32),
            pltpu.VMEM((_CH,), jnp.int32),
            pltpu.VMEM((_CH, d), jnp.float32),
            pltpu.VMEM((_CH, d), jnp.float32),
            pltpu.SemaphoreType.DMA,
            pltpu.SemaphoreType.DMA,
        ],
    )
    def gather_k(xl_hbm, xr_hbm, s_hbm, d_hbm, gl_hbm, gr_hbm,
                 si_v, di_v, ra_v, rb_v, sem_a, sem_b):
        wid = lax.axis_index("s") * nc + lax.axis_index("c")
        base = wid * bw

        def body(j, carry):
            off = base + j * _CH
            pltpu.sync_copy(s_hbm.at[pl.ds(off, _CH)], si_v)
            pltpu.sync_copy(d_hbm.at[pl.ds(off, _CH)], di_v)
            ca = pltpu.async_copy(xl_hbm.at[si_v], ra_v, sem_a)
            cb = pltpu.async_copy(xr_hbm.at[di_v], rb_v, sem_b)
            ca.wait()
            cb.wait()
            pltpu.sync_copy(ra_v, gl_hbm.at[pl.ds(off, _CH)])
            pltpu.sync_copy(rb_v, gr_hbm.at[pl.ds(off, _CH)])
            return carry

        lax.fori_loop(0, chunks, body, 0)

    return gather_k(xl, xr, sidx, didx)


def _mm(x, wt, b, bm=512):
    m, k = x.shape
    mo = wt.shape[1]
    bm = min(bm, max(8, (m + 7) // 8 * 8))

    def kern(xr, wr, br, orr):
        orr[...] = jnp.dot(xr[...], wr[...],
                           preferred_element_type=jnp.float32) + br[...]

    return pl.pallas_call(
        kern,
        grid=(pl.cdiv(m, bm),),
        in_specs=[
            pl.BlockSpec((bm, k), lambda i: (i, 0)),
            pl.BlockSpec((k, mo), lambda i: (0, 0)),
            pl.BlockSpec((1, mo), lambda i: (0, 0)),
        ],
        out_specs=pl.BlockSpec((bm, mo), lambda i: (i, 0)),
        out_shape=jax.ShapeDtypeStruct((m, mo), jnp.float32),
    )(x, wt, b.reshape(1, -1))


def _edge_stage(gl, gr, ge, att, hmap, hmapt, c_out, bm=512):
    ne, c = gl.shape
    has_ge = ge is not None

    def kern(*refs):
        if has_ge:
            glr, grr, ger, ar, hr, htr, wr, pr = refs
            msg = glr[...] + grr[...] + ger[...]
        else:
            glr, grr, ar, hr, htr, wr, pr = refs
            msg = glr[...] + grr[...]
        t = jnp.where(msg > 0, msg, 0.2 * msg) * ar[...]
        logits = jnp.dot(t, hr[...], preferred_element_type=jnp.float32)
        p = jnp.exp(logits)
        pe = jnp.dot(p, htr[...], preferred_element_type=jnp.float32)
        wr[...] = (glr[...] * pe)[:, :c_out]
        pr[...] = p

    row = pl.BlockSpec((bm, c), lambda i: (i, 0))
    in_specs = [row, row]
    args = [gl, gr]
    if has_ge:
        in_specs.append(row)
        args.append(ge)
    in_specs += [
        pl.BlockSpec((1, c), lambda i: (0, 0)),
        pl.BlockSpec((c, 128), lambda i: (0, 0)),
        pl.BlockSpec((128, c), lambda i: (0, 0)),
    ]
    args += [att, hmap, hmapt]
    return pl.pallas_call(
        kern,
        grid=(pl.cdiv(ne, bm),),
        in_specs=in_specs,
        out_specs=[
            pl.BlockSpec((bm, c_out), lambda i: (i, 0)),
            pl.BlockSpec((bm, 128), lambda i: (i, 0)),
        ],
        out_shape=[
            jax.ShapeDtypeStruct((ne, c_out), jnp.float32),
            jax.ShapeDtypeStruct((ne, 128), jnp.float32),
        ],
    )(*args)


def _normalize(num, s_pad, hmapt, bias, bm=512):
    n, c = num.shape

    def kern(nr, sr, htr, br, orr):
        sexp = jnp.dot(sr[...], htr[...], preferred_element_type=jnp.float32)
        o = nr[...] / (sexp + _EPS) + br[...]
        orr[...] = jnp.where(o > 0, o, jnp.exp(jnp.minimum(o, 0.0)) - 1.0)

    return pl.pallas_call(
        kern,
        grid=(pl.cdiv(n, bm),),
        in_specs=[
            pl.BlockSpec((bm, c), lambda i: (i, 0)),
            pl.BlockSpec((bm, 128), lambda i: (i, 0)),
            pl.BlockSpec((128, c), lambda i: (0, 0)),
            pl.BlockSpec((1, c), lambda i: (0, 0)),
        ],
        out_specs=pl.BlockSpec((bm, c), lambda i: (i, 0)),
        out_shape=jax.ShapeDtypeStruct((n, c), jnp.float32),
    )(num, s_pad, hmapt, bias.reshape(1, -1))


def _scatter_mean(data, idx, num_segments):
    s = jax.ops.segment_sum(data, idx, num_segments=num_segments)
    c = jax.ops.segment_sum(jnp.ones((idx.shape[0],), dtype=data.dtype), idx,
                            num_segments=num_segments)
    c = jnp.clip(c, 1.0, None)
    if data.ndim > 1:
        c = c.reshape((-1,) + (1,) * (data.ndim - 1))
    return s / c


def _gat_layer(x, src, dst, wl, bl, wr, br, att, bias, heads, out_ch,
               edge_feat=None, we=None):
    n = x.shape[0]
    c = heads * out_ch
    ne = src.shape[0] + n
    bpad = ((ne + 4095) // 4096) * 4096
    npad = bpad - ne
    loop = jnp.arange(n, dtype=src.dtype)
    zpad = jnp.zeros((npad,), dtype=src.dtype)
    if edge_feat is not None:
        loop_attr = _scatter_mean(edge_feat, dst, n)
        edge_feat = jnp.concatenate(
            [edge_feat, loop_attr,
             jnp.zeros((npad, edge_feat.shape[1]), edge_feat.dtype)], axis=0)
    src_g = jnp.minimum(jnp.concatenate([src, loop, zpad]), n - 1)
    dst_g = jnp.minimum(jnp.concatenate([dst, loop, zpad]), n - 1)
    dst_s = jnp.concatenate([dst, loop, jnp.full((npad,), n, src.dtype)])

    cp = max(c, 128)

    def cpad(m2):
        return jnp.pad(m2, ((0, 0), (0, cp - c)))

    xl = _mm(x, cpad(wl.T), jnp.pad(bl, (0, cp - c)))
    xr = _mm(x, cpad(wr.T), jnp.pad(br, (0, cp - c)))
    ge = _mm(edge_feat, cpad(we.T), jnp.zeros((cp,), jnp.float32)) \
        if edge_feat is not None else None

    gl, gr = _sc_pair_gather(xl, xr, src_g, dst_g)

    hmap_e = (jnp.arange(cp)[:, None] // out_ch
              == jnp.arange(128)[None, :]).astype(jnp.float32)
    hmapt_n = (jnp.arange(128)[:, None]
               == jnp.arange(c)[None, :] // out_ch).astype(jnp.float32)
    att1 = jnp.pad(att.reshape(1, c), ((0, 0), (0, cp - c)))

    w, p_pad = _edge_stage(gl, gr, ge, att1, hmap_e, hmap_e.T, c)
    p = p_pad[:, :heads]
    s = jax.ops.segment_sum(p, dst_s, num_segments=n)
    num = jax.ops.segment_sum(w, dst_s, num_segments=n)
    s_pad = jnp.pad(s, ((0, 0), (0, 128 - heads)))
    return _normalize(num, s_pad, hmapt_n, bias)


def _rg_edges(edge_index, group_idx, num_groups):
    ge = group_idx[edge_index]
    valid = ge[0] != ge[1]
    sent = num_groups * num_groups
    key = jnp.where(valid, ge[0] * num_groups + ge[1], sent)
    key = jnp.sort(key)
    first = jnp.concatenate([jnp.ones((1,), dtype=jnp.bool_),
                             key[1:] != key[:-1]])
    keep = first & (key < sent)
    srcg = jnp.where(keep, key // num_groups, num_groups)
    dstg = jnp.where(keep, key % num_groups, num_groups)
    return jnp.stack([srcg, dstg], axis=0), jnp.any(keep)


def _mlp_head(pooled, w1t, b1, w2t_pad, b2_pad):
    def kern(pr, w1r, b1r, w2r, b2r, orr):
        o = jnp.dot(pr[...], w1r[...], preferred_element_type=jnp.float32)
        o = jnp.maximum(o + b1r[...], 0.0)
        orr[...] = jnp.dot(o, w2r[...],
                           preferred_element_type=jnp.float32) + b2r[...]

    m, k = pooled.shape
    ko = w1t.shape[1]
    return pl.pallas_call(
        kern,
        grid=(1,),
        in_specs=[
            pl.BlockSpec((m, k), lambda i: (0, 0)),
            pl.BlockSpec((k, ko), lambda i: (0, 0)),
            pl.BlockSpec((1, ko), lambda i: (0, 0)),
            pl.BlockSpec((ko, 128), lambda i: (0, 0)),
            pl.BlockSpec((1, 128), lambda i: (0, 0)),
        ],
        out_specs=pl.BlockSpec((m, 128), lambda i: (0, 0)),
        out_shape=jax.ShapeDtypeStruct((m, 128), jnp.float32),
    )(pooled, w1t, b1.reshape(1, -1), w2t_pad, b2_pad.reshape(1, -1))


def kernel(x, edge_index, edge_attr, batch, pharma_index, ew, eb, g1_Wl, g1_bl, g1_Wr, g1_br, g1_att, g1_We, g1_bias, g2_Wl, g2_bl, g2_Wr, g2_br, g2_att, g2_We, g2_bias, g3_Wl, g3_bl, g3_Wr, g3_br, g3_att, g3_bias, g4_Wl, g4_bl, g4_Wr, g4_br, g4_att, g4_bias, l1_W, l1_b, l2_W, l2_b):
    n = x.shape[0]
    g = 2000
    b_sz = 64
    h = 64

    src = edge_index[0]
    dst = edge_index[1]
    ea = _mm(edge_attr, ew.T, eb)

    h1 = _gat_layer(x, src, dst, g1_Wl, g1_bl, g1_Wr, g1_br, g1_att, g1_bias,
                    4, h, ea, g1_We)
    h2 = _gat_layer(h1, src, dst, g2_Wl, g2_bl, g2_Wr, g2_br, g2_att, g2_bias,
                    1, h, ea, g2_We)

    grouped = _scatter_mean(h2, pharma_index, g)
    red_batch = _scatter_mean(batch.astype(jnp.float32), pharma_index,
                              g).astype(jnp.int32)
    nei, has_edges = _rg_edges(edge_index, pharma_index, g)

    def with_edges(gz):
        z1 = _gat_layer(gz, nei[0], nei[1], g3_Wl, g3_bl, g3_Wr, g3_br,
                        g3_att, g3_bias, 1, h)
        return _gat_layer(z1, nei[0], nei[1], g4_Wl, g4_bl, g4_Wr, g4_br,
                          g4_att, g4_bias, 1, h)

    z = jax.lax.cond(has_edges, with_edges, lambda gz: gz, grouped)
    pooled = _scatter_mean(z, red_batch, b_sz)

    w2t_pad = jnp.pad(l2_W.T, ((0, 0), (0, 127)))
    b2_pad = jnp.pad(l2_b, (0, 127))
    out = _mlp_head(pooled, l1_W.T, l1_b, w2t_pad, b2_pad)
    return out[:, :1]
